# baseline (speedup 1.0000x reference)
import functools

import jax
import jax.numpy as jnp
import numpy as np
from jax import lax
from jax.experimental import pallas as pl
from jax.experimental.pallas import tpu as pltpu
from jax.experimental.pallas import tpu_sc as plsc

_COULOMB_FACTOR = 14.399645478425668
_MAGIC = np.int32(0x5F3759DF)
_SPLIT_SEG = 10


def _pair_energy_sc(x, y, z, q, num_graphs, atoms, seg_start):
    L = 16
    n_chunks = atoms // L
    mesh = plsc.VectorSubcoreMesh(core_axis_name="c", subcore_axis_name="s")

    @functools.partial(
        pl.kernel,
        out_type=jax.ShapeDtypeStruct((num_graphs, L), jnp.float32),
        mesh=mesh,
        scratch_types=[
            pltpu.VMEM((atoms + L,), jnp.float32),
            pltpu.VMEM((atoms + L,), jnp.float32),
            pltpu.VMEM((atoms + L,), jnp.float32),
            pltpu.VMEM((atoms + L,), jnp.float32),
            pltpu.VMEM((L,), jnp.float32),
            pltpu.SemaphoreType.DMA,
        ],
    )
    def body(x_hbm, y_hbm, z_hbm, q_hbm, out_hbm, xv, yv, zv, qv, outv, sem):
        wid = lax.axis_index("s") * 2 + lax.axis_index("c")
        base = wid * atoms
        copies = [
            pltpu.async_copy(src.at[pl.ds(base, atoms)],
                             dst.at[pl.ds(0, atoms)], sem)
            for src, dst in ((x_hbm, xv), (y_hbm, yv), (z_hbm, zv),
                             (q_hbm, qv))
        ]
        for cp in copies:
            cp.wait()

        lane = lax.iota(jnp.int32, L)
        eps = jnp.float32(1e-6)
        half = jnp.float32(0.5)
        three_half = jnp.float32(1.5)

        def pair_chunk(i_vec, xi, yi, zi, qi, jc, acc, masked):
            off = jc * L
            xj = xv[pl.ds(off, L)]
            yj = yv[pl.ds(off, L)]
            zj = zv[pl.ds(off, L)]
            qj = qv[pl.ds(off, L)]
            dx = xi - xj
            dy = yi - yj
            dz = zi - zj
            s = dx * dx + dy * dy + dz * dz
            s_bits = lax.bitcast_convert_type(s, jnp.int32)
            r = lax.bitcast_convert_type(_MAGIC - (s_bits >> 1), jnp.float32)
            h = half * s
            r = r * (three_half - h * r * r)
            r = r * (three_half - h * r * r)
            c = qi * qj * r
            if masked:
                j_idx = lane + off
                c = jnp.where(j_idx > i_vec, c, jnp.float32(0.0))
            return acc + c

        def make_seg_body(seg):
            def seg_body(i, acc):
                i_vec = jnp.full((L,), i, dtype=jnp.int32)
                xi = jnp.full((L,), xv[pl.ds(i, L)][0], dtype=jnp.float32) + eps
                yi = jnp.full((L,), yv[pl.ds(i, L)][0], dtype=jnp.float32) + eps
                zi = jnp.full((L,), zv[pl.ds(i, L)][0], dtype=jnp.float32) + eps
                qi = jnp.full((L,), qv[pl.ds(i, L)][0], dtype=jnp.float32)
                for jc in range(seg, n_chunks):
                    acc = pair_chunk(i_vec, xi, yi, zi, qi, jc, acc,
                                     masked=jc == seg)
                return acc
            return seg_body

        acc = jnp.zeros((L,), jnp.float32)
        for seg in range(seg_start, n_chunks):
            acc = lax.fori_loop(seg * L, (seg + 1) * L, make_seg_body(seg),
                                acc, unroll=False)

        dnums = lax.GatherDimensionNumbers(
            offset_dims=(), collapsed_slice_dims=(0,), start_index_map=(0,))
        for stride in (8, 4, 2, 1):
            idx = lax.iota(jnp.int32, L) ^ stride
            shuffled = lax.gather(
                acc, idx[:, None], dimension_numbers=dnums, slice_sizes=(1,),
                mode=lax.GatherScatterMode.PROMISE_IN_BOUNDS)
            acc = acc + shuffled

        outv[...] = acc * jnp.float32(_COULOMB_FACTOR)
        pltpu.sync_copy(outv, out_hbm.at[wid])

    return body(x, y, z, q)


def _pair_energy_tc(x, y, z, q, num_graphs, atoms, rows):
    mols_per_step = 4

    def body(x_ref, y_ref, z_ref, q_ref, o_ref):
        eps = jnp.float32(1e-6)
        row_ids = lax.broadcasted_iota(jnp.int32, (rows, atoms), 0)
        col_ids = lax.broadcasted_iota(jnp.int32, (rows, atoms), 1)
        mask = col_ids > row_ids
        for m in range(mols_per_step):
            xr = x_ref[m, 0, :]
            yr = y_ref[m, 0, :]
            zr = z_ref[m, 0, :]
            qr = q_ref[m, 0, :]
            dx = xr[:rows].reshape(rows, 1) - xr.reshape(1, atoms) + eps
            dy = yr[:rows].reshape(rows, 1) - yr.reshape(1, atoms) + eps
            dz = zr[:rows].reshape(rows, 1) - zr.reshape(1, atoms) + eps
            s = dx * dx + dy * dy + dz * dz
            r = lax.rsqrt(s)
            qq = qr[:rows].reshape(rows, 1) * qr.reshape(1, atoms)
            e = jnp.where(mask, qq * r, jnp.float32(0.0))
            acc8 = jnp.zeros((8, 128), jnp.float32)
            for rr in range(0, rows, 8):
                for cc in range(0, atoms, 128):
                    acc8 = acc8 + e[rr:rr + 8, cc:cc + 128]
            tot = jnp.sum(acc8) * jnp.float32(_COULOMB_FACTOR)
            o_ref[m, 0, :] = jnp.full((128,), tot, dtype=jnp.float32)

    spec = pl.BlockSpec((mols_per_step, 1, atoms), lambda g: (g, 0, 0))
    return pl.pallas_call(
        body,
        grid=(num_graphs // mols_per_step,),
        in_specs=[spec, spec, spec, spec],
        out_specs=pl.BlockSpec((mols_per_step, 1, 128), lambda g: (g, 0, 0)),
        out_shape=jax.ShapeDtypeStruct((num_graphs, 1, 128), jnp.float32),
    )(x, y, z, q)


def kernel(pos, charges, ptr):
    num_graphs = ptr.shape[0] - 1
    atoms = pos.shape[0] // num_graphs
    x = pos[:, 0]
    y = pos[:, 1]
    z = pos[:, 2]
    q = charges[:, 0]
    sc_out = _pair_energy_sc(x, y, z, q, num_graphs, atoms, _SPLIT_SEG)
    xb = x.reshape(num_graphs, 1, atoms)
    yb = y.reshape(num_graphs, 1, atoms)
    zb = z.reshape(num_graphs, 1, atoms)
    qb = q.reshape(num_graphs, 1, atoms)
    tc_out = _pair_energy_tc(xb, yb, zb, qb, num_graphs, atoms,
                             _SPLIT_SEG * 16)
    return sc_out[:, :1] + tc_out[:, 0, :1]


# speedup vs baseline: 1.0054x; 1.0054x over previous
import functools

import jax
import jax.numpy as jnp
import numpy as np
from jax import lax
from jax.experimental import pallas as pl
from jax.experimental.pallas import tpu as pltpu
from jax.experimental.pallas import tpu_sc as plsc

_COULOMB_FACTOR = 14.399645478425668
_MAGIC = np.int32(0x5F3759DF)
_SPLIT_SEG = 9


def _pair_energy_sc(x, y, z, q, num_graphs, atoms, seg_start):
    L = 16
    n_chunks = atoms // L
    mesh = plsc.VectorSubcoreMesh(core_axis_name="c", subcore_axis_name="s")

    @functools.partial(
        pl.kernel,
        out_type=jax.ShapeDtypeStruct((num_graphs, L), jnp.float32),
        mesh=mesh,
        scratch_types=[
            pltpu.VMEM((atoms + L,), jnp.float32),
            pltpu.VMEM((atoms + L,), jnp.float32),
            pltpu.VMEM((atoms + L,), jnp.float32),
            pltpu.VMEM((atoms + L,), jnp.float32),
            pltpu.VMEM((L,), jnp.float32),
            pltpu.SemaphoreType.DMA,
        ],
    )
    def body(x_hbm, y_hbm, z_hbm, q_hbm, out_hbm, xv, yv, zv, qv, outv, sem):
        wid = lax.axis_index("s") * 2 + lax.axis_index("c")
        base = wid * atoms
        copies = [
            pltpu.async_copy(src.at[pl.ds(base, atoms)],
                             dst.at[pl.ds(0, atoms)], sem)
            for src, dst in ((x_hbm, xv), (y_hbm, yv), (z_hbm, zv),
                             (q_hbm, qv))
        ]
        for cp in copies:
            cp.wait()

        lane = lax.iota(jnp.int32, L)
        eps = jnp.float32(1e-6)
        half = jnp.float32(0.5)
        three_half = jnp.float32(1.5)

        def pair_chunk(i_vec, xi, yi, zi, qi, jc, acc, masked):
            off = jc * L
            xj = xv[pl.ds(off, L)]
            yj = yv[pl.ds(off, L)]
            zj = zv[pl.ds(off, L)]
            qj = qv[pl.ds(off, L)]
            dx = xi - xj
            dy = yi - yj
            dz = zi - zj
            s = dx * dx + dy * dy + dz * dz
            s_bits = lax.bitcast_convert_type(s, jnp.int32)
            r = lax.bitcast_convert_type(_MAGIC - (s_bits >> 1), jnp.float32)
            h = half * s
            r = r * (three_half - h * r * r)
            r = r * (three_half - h * r * r)
            c = qi * qj * r
            if masked:
                j_idx = lane + off
                c = jnp.where(j_idx > i_vec, c, jnp.float32(0.0))
            return acc + c

        def make_seg_body(seg):
            def seg_body(i, acc):
                i_vec = jnp.full((L,), i, dtype=jnp.int32)
                xi = jnp.full((L,), xv[pl.ds(i, L)][0], dtype=jnp.float32) + eps
                yi = jnp.full((L,), yv[pl.ds(i, L)][0], dtype=jnp.float32) + eps
                zi = jnp.full((L,), zv[pl.ds(i, L)][0], dtype=jnp.float32) + eps
                qi = jnp.full((L,), qv[pl.ds(i, L)][0], dtype=jnp.float32)
                for jc in range(seg, n_chunks):
                    acc = pair_chunk(i_vec, xi, yi, zi, qi, jc, acc,
                                     masked=jc == seg)
                return acc
            return seg_body

        acc = jnp.zeros((L,), jnp.float32)
        for seg in range(seg_start, n_chunks):
            acc = lax.fori_loop(seg * L, (seg + 1) * L, make_seg_body(seg),
                                acc, unroll=False)

        dnums = lax.GatherDimensionNumbers(
            offset_dims=(), collapsed_slice_dims=(0,), start_index_map=(0,))
        for stride in (8, 4, 2, 1):
            idx = lax.iota(jnp.int32, L) ^ stride
            shuffled = lax.gather(
                acc, idx[:, None], dimension_numbers=dnums, slice_sizes=(1,),
                mode=lax.GatherScatterMode.PROMISE_IN_BOUNDS)
            acc = acc + shuffled

        outv[...] = acc * jnp.float32(_COULOMB_FACTOR)
        pltpu.sync_copy(outv, out_hbm.at[wid])

    return body(x, y, z, q)


def _pair_energy_tc(x, y, z, q, num_graphs, atoms, rows):
    mols_per_step = 4

    def body(x_ref, y_ref, z_ref, q_ref, o_ref):
        eps = jnp.float32(1e-6)
        row_ids = lax.broadcasted_iota(jnp.int32, (rows, atoms), 0)
        col_ids = lax.broadcasted_iota(jnp.int32, (rows, atoms), 1)
        mask = col_ids > row_ids
        for m in range(mols_per_step):
            xr = x_ref[m, 0, :]
            yr = y_ref[m, 0, :]
            zr = z_ref[m, 0, :]
            qr = q_ref[m, 0, :]
            dx = xr[:rows].reshape(rows, 1) - xr.reshape(1, atoms) + eps
            dy = yr[:rows].reshape(rows, 1) - yr.reshape(1, atoms) + eps
            dz = zr[:rows].reshape(rows, 1) - zr.reshape(1, atoms) + eps
            s = dx * dx + dy * dy + dz * dz
            r = lax.rsqrt(s)
            qq = qr[:rows].reshape(rows, 1) * qr.reshape(1, atoms)
            e = jnp.where(mask, qq * r, jnp.float32(0.0))
            acc8 = jnp.zeros((8, 128), jnp.float32)
            for rr in range(0, rows, 8):
                for cc in range(0, atoms, 128):
                    acc8 = acc8 + e[rr:rr + 8, cc:cc + 128]
            tot = jnp.sum(acc8) * jnp.float32(_COULOMB_FACTOR)
            o_ref[m, 0, :] = jnp.full((128,), tot, dtype=jnp.float32)

    spec = pl.BlockSpec((mols_per_step, 1, atoms), lambda g: (g, 0, 0))
    return pl.pallas_call(
        body,
        grid=(num_graphs // mols_per_step,),
        in_specs=[spec, spec, spec, spec],
        out_specs=pl.BlockSpec((mols_per_step, 1, 128), lambda g: (g, 0, 0)),
        out_shape=jax.ShapeDtypeStruct((num_graphs, 1, 128), jnp.float32),
    )(x, y, z, q)


def kernel(pos, charges, ptr):
    num_graphs = ptr.shape[0] - 1
    atoms = pos.shape[0] // num_graphs
    x = pos[:, 0]
    y = pos[:, 1]
    z = pos[:, 2]
    q = charges[:, 0]
    sc_out = _pair_energy_sc(x, y, z, q, num_graphs, atoms, _SPLIT_SEG)
    xb = x.reshape(num_graphs, 1, atoms)
    yb = y.reshape(num_graphs, 1, atoms)
    zb = z.reshape(num_graphs, 1, atoms)
    qb = q.reshape(num_graphs, 1, atoms)
    tc_out = _pair_energy_tc(xb, yb, zb, qb, num_graphs, atoms,
                             _SPLIT_SEG * 16)
    return sc_out[:, :1] + tc_out[:, 0, :1]
